# baseline (speedup 1.0000x reference)
import jax
import jax.numpy as jnp
from jax.experimental import pallas as pl
from jax.experimental.pallas import tpu as pltpu

_B = 4
_L = 8192
_H = 896
_H4 = 224
_K = 6
_BL = 2048
_NS = _B * _L // _BL
_JB = _L // _BL


def _body(b2_ref, h_ref, w1_ref, b1_ref, w2_ref, s_ref, idx_ref, mask_ref):
    i = pl.program_id(0)
    x = jnp.dot(h_ref[0], w1_ref[...], preferred_element_type=jnp.float32)
    x = jnp.maximum(x + b1_ref[...], 0.0)
    s = jax.lax.dot_general(w2_ref[...], x, (((0,), (1,)), ((), ())),
                            preferred_element_type=jnp.float32)
    row = i // _JB
    s_ref[pl.ds(row, 1), pl.ds((i % _JB) * _BL, _BL)] = s + b2_ref[0]

    @pl.when(i % _JB == _JB - 1)
    def _row_epilogue():
        cur = s_ref[pl.ds(row, 1), :]
        col = jax.lax.broadcasted_iota(jnp.int32, (1, _L), 1)
        lane = jax.lax.broadcasted_iota(jnp.int32, (1, 128), 1)
        mask_acc = jnp.zeros((1, _L), jnp.float32)
        idx_acc = jnp.zeros((1, 128), jnp.int32)
        for k in range(_K):
            m = jnp.max(cur, axis=1, keepdims=True)
            idx = jnp.min(jnp.where(cur == m, col, _L), axis=1, keepdims=True)
            onehot = col == idx
            mask_acc = jnp.where(onehot, 1.0, mask_acc)
            cur = jnp.where(onehot, -jnp.inf, cur)
            idx_acc = jnp.where(lane == k, idx, idx_acc)
        mask_ref[pl.ds(row, 1), :] = mask_acc
        idx_ref[pl.ds(row, 1), :] = idx_acc


@jax.jit
def kernel(hidden_states, W1, b1, W2, b2):
    b, l, h = hidden_states.shape
    scores, idx128, mask = pl.pallas_call(
        _body,
        grid=(_NS,),
        in_specs=[
            pl.BlockSpec(memory_space=pltpu.SMEM),
            pl.BlockSpec((1, _BL, _H), lambda i: (i, 0, 0)),
            pl.BlockSpec((_H, _H4), lambda i: (0, 0)),
            pl.BlockSpec((1, _H4), lambda i: (0, 0)),
            pl.BlockSpec((_H4, 1), lambda i: (0, 0)),
        ],
        out_specs=(
            pl.BlockSpec((_B, _L), lambda i: (0, 0)),
            pl.BlockSpec((_B, 128), lambda i: (0, 0)),
            pl.BlockSpec((_B, _L), lambda i: (0, 0)),
        ),
        out_shape=(
            jax.ShapeDtypeStruct((_B, _L), jnp.float32),
            jax.ShapeDtypeStruct((_B, 128), jnp.int32),
            jax.ShapeDtypeStruct((_B, _L), jnp.float32),
        ),
        compiler_params=pltpu.CompilerParams(
            dimension_semantics=("arbitrary",)),
    )(b2, hidden_states.reshape(_NS, _BL, _H), W1, b1.reshape(1, _H4), W2)
    return scores, idx128[:, :_K], mask


# speedup vs baseline: 1.1209x; 1.1209x over previous
import jax
import jax.numpy as jnp
from jax.experimental import pallas as pl
from jax.experimental.pallas import tpu as pltpu

_B = 4
_L = 8192
_H = 896
_H4 = 224
_K = 6
_BL = 2048
_NS = _B * _L // _BL
_JB = _L // _BL


def _body(b2_ref, h_ref, w1_ref, b1_ref, w2_ref, s_ref, idx_ref, mask_ref):
    i = pl.program_id(0)
    x = jnp.dot(h_ref[0], w1_ref[...], preferred_element_type=jnp.float32)
    x = jnp.maximum(x + b1_ref[...], 0.0)
    s = jax.lax.dot_general(w2_ref[...], x, (((0,), (1,)), ((), ())),
                            preferred_element_type=jnp.float32)
    s_ref[pl.ds(i // _JB, 1), pl.ds((i % _JB) * _BL, _BL)] = s + b2_ref[0]

    @pl.when(i == _NS - 1)
    def _epilogue():
        cur = s_ref[...]
        col = jax.lax.broadcasted_iota(jnp.int32, (_B, _L), 1)
        lane = jax.lax.broadcasted_iota(jnp.int32, (_B, 128), 1)
        mask_acc = jnp.zeros((_B, _L), jnp.float32)
        idx_acc = jnp.zeros((_B, 128), jnp.int32)
        for k in range(_K):
            m = jnp.max(cur, axis=1, keepdims=True)
            idx = jnp.min(jnp.where(cur == m, col, _L), axis=1, keepdims=True)
            onehot = col == idx
            mask_acc = jnp.where(onehot, 1.0, mask_acc)
            cur = jnp.where(onehot, -jnp.inf, cur)
            idx_acc = jnp.where(lane == k, idx, idx_acc)
        mask_ref[...] = mask_acc
        idx_ref[...] = idx_acc


@jax.jit
def kernel(hidden_states, W1, b1, W2, b2):
    b, l, h = hidden_states.shape
    scores, idx128, mask = pl.pallas_call(
        _body,
        grid=(_NS,),
        in_specs=[
            pl.BlockSpec(memory_space=pltpu.SMEM),
            pl.BlockSpec((1, _BL, _H), lambda i: (i, 0, 0)),
            pl.BlockSpec((_H, _H4), lambda i: (0, 0)),
            pl.BlockSpec((1, _H4), lambda i: (0, 0)),
            pl.BlockSpec((_H4, 1), lambda i: (0, 0)),
        ],
        out_specs=(
            pl.BlockSpec((_B, _L), lambda i: (0, 0)),
            pl.BlockSpec((_B, 128), lambda i: (0, 0)),
            pl.BlockSpec((_B, _L), lambda i: (0, 0)),
        ),
        out_shape=(
            jax.ShapeDtypeStruct((_B, _L), jnp.float32),
            jax.ShapeDtypeStruct((_B, 128), jnp.int32),
            jax.ShapeDtypeStruct((_B, _L), jnp.float32),
        ),
        compiler_params=pltpu.CompilerParams(
            dimension_semantics=("arbitrary",)),
    )(b2, hidden_states.reshape(_NS, _BL, _H), W1, b1.reshape(1, _H4), W2)
    return scores, idx128[:, :_K], mask


# speedup vs baseline: 1.1808x; 1.0535x over previous
import jax
import jax.numpy as jnp
from jax.experimental import pallas as pl
from jax.experimental.pallas import tpu as pltpu

_B = 4
_L = 8192
_H = 896
_H4 = 224
_K = 6
_BL = 4096
_NS = _B * _L // _BL
_JB = _L // _BL


def _body(b2_ref, h_ref, w1_ref, b1_ref, w2_ref, s_ref, idx_ref, mask_ref):
    i = pl.program_id(0)
    x = jnp.dot(h_ref[0], w1_ref[...], preferred_element_type=jnp.float32)
    x = jnp.maximum(x + b1_ref[...], 0.0)
    s = jax.lax.dot_general(w2_ref[...], x, (((0,), (1,)), ((), ())),
                            preferred_element_type=jnp.float32)
    s_ref[pl.ds(i // _JB, 1), pl.ds((i % _JB) * _BL, _BL)] = s + b2_ref[0]

    @pl.when(i == _NS - 1)
    def _epilogue():
        cur = s_ref[...]
        col = jax.lax.broadcasted_iota(jnp.int32, (_B, _L), 1)
        lane = jax.lax.broadcasted_iota(jnp.int32, (_B, 128), 1)
        mask_acc = jnp.zeros((_B, _L), jnp.float32)
        idx_acc = jnp.zeros((_B, 128), jnp.int32)
        for k in range(_K):
            m = jnp.max(cur, axis=1, keepdims=True)
            idx = jnp.min(jnp.where(cur == m, col, _L), axis=1, keepdims=True)
            onehot = col == idx
            mask_acc = jnp.where(onehot, 1.0, mask_acc)
            cur = jnp.where(onehot, -jnp.inf, cur)
            idx_acc = jnp.where(lane == k, idx, idx_acc)
        mask_ref[...] = mask_acc
        idx_ref[...] = idx_acc


@jax.jit
def kernel(hidden_states, W1, b1, W2, b2):
    b, l, h = hidden_states.shape
    scores, idx128, mask = pl.pallas_call(
        _body,
        grid=(_NS,),
        in_specs=[
            pl.BlockSpec(memory_space=pltpu.SMEM),
            pl.BlockSpec((1, _BL, _H), lambda i: (i, 0, 0)),
            pl.BlockSpec((_H, _H4), lambda i: (0, 0)),
            pl.BlockSpec((1, _H4), lambda i: (0, 0)),
            pl.BlockSpec((_H4, 1), lambda i: (0, 0)),
        ],
        out_specs=(
            pl.BlockSpec((_B, _L), lambda i: (0, 0)),
            pl.BlockSpec((_B, 128), lambda i: (0, 0)),
            pl.BlockSpec((_B, _L), lambda i: (0, 0)),
        ),
        out_shape=(
            jax.ShapeDtypeStruct((_B, _L), jnp.float32),
            jax.ShapeDtypeStruct((_B, 128), jnp.int32),
            jax.ShapeDtypeStruct((_B, _L), jnp.float32),
        ),
        compiler_params=pltpu.CompilerParams(
            dimension_semantics=("arbitrary",)),
    )(b2, hidden_states.reshape(_NS, _BL, _H), W1.astype(jnp.bfloat16),
      b1.reshape(1, _H4), W2)
    return scores, idx128[:, :_K], mask
